# baseline (speedup 1.0000x reference)
import math

import jax
import jax.numpy as jnp
from jax import lax
from jax.experimental import pallas as pl
from jax.experimental.pallas import tpu as pltpu

H = 4
SPARSITY_RATIO = 0.1
MEMORY_PRESSURE = 0.5
THRESH = SPARSITY_RATIO * (1.0 + MEMORY_PRESSURE)
NEG = -1e30
BQ = 512

_DNT = (((1,), (1,)), ((), ()))


def _mmt(x, w):
    return lax.dot_general(x, w, _DNT, preferred_element_type=jnp.float32)


def _fused_kernel(k_ref, v_ref, q_ref, wk_ref, wv_ref, wq_ref,
                  w1_ref, b1_ref, w2_ref, b2_ref, ipb_ref, ow_ref, ob_ref,
                  out_ref, kp_s, vp_s, imp_s):
    i = pl.program_id(0)
    nblk = pl.num_programs(0) // 2
    d = k_ref.shape[1]
    hd = d // H

    @pl.when(i < nblk)
    def _proj_phase():
        rows = pl.ds(i * BQ, BQ)
        kp = _mmt(k_ref[...], wk_ref[...]) + ipb_ref[1]
        vp = _mmt(v_ref[...], wv_ref[...]) + ipb_ref[2]
        kp_s[rows, :] = kp.astype(jnp.bfloat16)
        vp_s[rows, :] = vp.astype(jnp.bfloat16)
        hid = jnp.maximum(_mmt(q_ref[...], w1_ref[...]) + b1_ref[...], 0.0)
        logit = _mmt(w2_ref[...], hid) + b2_ref[...]
        imp_s[:, pl.ds(i * BQ, BQ)] = jax.nn.sigmoid(logit)

    @pl.when(i >= nblk)
    def _attn_phase():
        j = i - nblk

        imp_row = imp_s[...]
        validk = (imp_row > THRESH).astype(jnp.float32)
        count = jnp.sum(validk)
        use_fb = count == 0.0
        fb_row = (lax.broadcasted_iota(jnp.int32, imp_row.shape, 1) < 32
                  ).astype(jnp.float32)
        validk = jnp.where(use_fb, fb_row, validk)
        kbias = (validk - 1.0) * (-NEG)

        impq = jnp.reshape(imp_s[0, pl.ds(j * BQ, BQ)], (BQ, 1))
        rows = lax.broadcasted_iota(jnp.int32, (BQ, 1), 0) + j * BQ
        validq = jnp.where(use_fb, (rows < 32).astype(jnp.float32),
                           (impq > THRESH).astype(jnp.float32))

        scale = 1.0 / math.sqrt(hd)
        qb = q_ref[...].astype(jnp.bfloat16)
        qp = ((_mmt(qb, wq_ref[...]) + ipb_ref[0]) * scale
              ).astype(jnp.bfloat16)
        acc = jnp.zeros(out_ref.shape, jnp.float32)
        for h in range(H):
            sl = slice(h * hd, (h + 1) * hd)
            s = lax.dot_general(qp[:, sl], kp_s[:, sl], _DNT,
                                preferred_element_type=jnp.float32)
            s = s + kbias
            m = jnp.max(s, axis=1, keepdims=True)
            p = jnp.exp(s - m)
            l = jnp.sum(p, axis=1, keepdims=True)
            ctx = lax.dot_general(p.astype(jnp.bfloat16), vp_s[:, sl],
                                  (((1,), (0,)), ((), ())),
                                  preferred_element_type=jnp.float32) / l
            acc = acc + _mmt(ctx, ow_ref[:, sl])
        out_ref[...] = (acc + ob_ref[...]) * validq


def kernel(q, k, v, W1, b1, W2, b2, in_proj_w, in_proj_b, out_w, out_b):
    batch, seq, d = q.shape
    n = batch * seq
    nblk = n // BQ
    dh = W1.shape[0]

    q2 = q.reshape(n, d)
    k2 = k.reshape(n, d).astype(jnp.bfloat16)
    v2 = v.reshape(n, d).astype(jnp.bfloat16)
    ipw = in_proj_w.astype(jnp.bfloat16)
    ipb = in_proj_b.reshape(3, 1, d)

    full = lambda shape: pl.BlockSpec(shape, lambda i: (0,) * len(shape))
    clamp = pl.BlockSpec((BQ, d), lambda i: (jnp.minimum(i, nblk - 1), 0))
    both = pl.BlockSpec((BQ, d),
                        lambda i: (jnp.where(i < nblk, i, i - nblk), 0))
    outsp = pl.BlockSpec((BQ, d),
                         lambda i: (jnp.maximum(i - nblk, 0), 0))
    ipw_at = lambda j: pl.BlockSpec((d, d), lambda i: (j, 0))

    out = pl.pallas_call(
        _fused_kernel,
        grid=(2 * nblk,),
        in_specs=[clamp, clamp, both,
                  ipw_at(1), ipw_at(2), ipw_at(0),
                  full((dh, d)), full((1, dh)), full((1, dh)), full((1, 1)),
                  full((3, 1, d)), full((d, d)), full((1, d))],
        out_specs=outsp,
        out_shape=jax.ShapeDtypeStruct((n, d), jnp.float32),
        scratch_shapes=[pltpu.VMEM((n, d), jnp.bfloat16),
                        pltpu.VMEM((n, d), jnp.bfloat16),
                        pltpu.VMEM((1, n), jnp.float32)],
    )(k2, v2, q2, ipw, ipw, ipw,
      W1, b1.reshape(1, dh), W2, b2.reshape(1, 1), ipb,
      out_w, out_b.reshape(1, d))

    return out.reshape(batch, seq, d)


# speedup vs baseline: 1.2150x; 1.2150x over previous
import math

import jax
import jax.numpy as jnp
from jax import lax
from jax.experimental import pallas as pl
from jax.experimental.pallas import tpu as pltpu

H = 4
SPARSITY_RATIO = 0.1
MEMORY_PRESSURE = 0.5
THRESH = SPARSITY_RATIO * (1.0 + MEMORY_PRESSURE)
NEG = -1e30
BQ = 512

_DNT = (((1,), (1,)), ((), ()))


def _mmt(x, w):
    return lax.dot_general(x, w, _DNT, preferred_element_type=jnp.float32)


def _fused_kernel(k_ref, v_ref, q_ref, wk_ref, wv_ref, wq_ref,
                  w1_ref, b1_ref, w2_ref, b2_ref, ipb_ref, ow_ref, ob_ref,
                  out_ref, kp_s, vp_s, imp_s):
    i = pl.program_id(0)
    nblk = pl.num_programs(0) // 2
    d = k_ref.shape[1]
    hd = d // H

    @pl.when(i < nblk)
    def _proj_phase():
        rows = pl.ds(i * BQ, BQ)
        kp = _mmt(k_ref[...], wk_ref[...]) + ipb_ref[1]
        vp = _mmt(v_ref[...], wv_ref[...]) + ipb_ref[2]
        kp_s[rows, :] = kp.astype(jnp.bfloat16)
        vp_s[rows, :] = vp.astype(jnp.bfloat16)
        hid = jnp.maximum(_mmt(q_ref[...], w1_ref[...]) + b1_ref[...], 0.0)
        logit = _mmt(w2_ref[...], hid) + b2_ref[...]
        imp_s[:, pl.ds(i * BQ, BQ)] = jax.nn.sigmoid(logit)

    @pl.when(i >= nblk)
    def _attn_phase():
        j = i - nblk

        imp_row = imp_s[...]
        validk = (imp_row > THRESH).astype(jnp.float32)
        count = jnp.sum(validk)
        use_fb = count == 0.0
        fb_row = (lax.broadcasted_iota(jnp.int32, imp_row.shape, 1) < 32
                  ).astype(jnp.float32)
        validk = jnp.where(use_fb, fb_row, validk)
        kbias = (validk - 1.0) * (-NEG)

        impq = jnp.reshape(imp_s[0, pl.ds(j * BQ, BQ)], (BQ, 1))
        rows = lax.broadcasted_iota(jnp.int32, (BQ, 1), 0) + j * BQ
        validq = jnp.where(use_fb, (rows < 32).astype(jnp.float32),
                           (impq > THRESH).astype(jnp.float32))

        scale = 1.0 / math.sqrt(hd)
        qp = ((_mmt(q_ref[...], wq_ref[...]) + ipb_ref[0]) * scale
              ).astype(jnp.bfloat16)
        acc = jnp.zeros(out_ref.shape, jnp.float32)
        for h in range(H):
            sl = slice(h * hd, (h + 1) * hd)
            s = lax.dot_general(qp[:, sl], kp_s[:, sl], _DNT,
                                preferred_element_type=jnp.float32)
            s = s + kbias
            m = jnp.max(s, axis=1, keepdims=True)
            p = jnp.exp(s - m)
            l = jnp.sum(p, axis=1, keepdims=True)
            ctx = lax.dot_general(p.astype(jnp.bfloat16), vp_s[:, sl],
                                  (((1,), (0,)), ((), ())),
                                  preferred_element_type=jnp.float32) / l
            acc = acc + _mmt(ctx, ow_ref[:, sl])
        out_ref[...] = (acc + ob_ref[...]) * validq


def kernel(q, k, v, W1, b1, W2, b2, in_proj_w, in_proj_b, out_w, out_b):
    batch, seq, d = q.shape
    n = batch * seq
    nblk = n // BQ
    dh = W1.shape[0]

    q2 = q.reshape(n, d)
    k2 = k.reshape(n, d)
    v2 = v.reshape(n, d)
    ipw = in_proj_w
    ipb = in_proj_b.reshape(3, 1, d)

    full = lambda shape: pl.BlockSpec(shape, lambda i: (0,) * len(shape))
    clamp = pl.BlockSpec((BQ, d), lambda i: (jnp.minimum(i, nblk - 1), 0))
    both = pl.BlockSpec((BQ, d),
                        lambda i: (jnp.where(i < nblk, i, i - nblk), 0))
    outsp = pl.BlockSpec((BQ, d),
                         lambda i: (jnp.maximum(i - nblk, 0), 0))
    ipw_at = lambda j: pl.BlockSpec((d, d), lambda i: (j, 0))

    out = pl.pallas_call(
        _fused_kernel,
        grid=(2 * nblk,),
        in_specs=[clamp, clamp, both,
                  ipw_at(1), ipw_at(2), ipw_at(0),
                  full((dh, d)), full((1, dh)), full((1, dh)), full((1, 1)),
                  full((3, 1, d)), full((d, d)), full((1, d))],
        out_specs=outsp,
        out_shape=jax.ShapeDtypeStruct((n, d), jnp.float32),
        scratch_shapes=[pltpu.VMEM((n, d), jnp.bfloat16),
                        pltpu.VMEM((n, d), jnp.bfloat16),
                        pltpu.VMEM((1, n), jnp.float32)],
    )(k2, v2, q2, ipw, ipw, ipw,
      W1, b1.reshape(1, dh), W2, b2.reshape(1, 1), ipb,
      out_w, out_b.reshape(1, d))

    return out.reshape(batch, seq, d)
